# Initial kernel scaffold; baseline (speedup 1.0000x reference)
import jax
import jax.numpy as jnp
from jax.experimental import pallas as pl


def kernel(x, supports, W1, b1, W2, b2):
    raise NotImplementedError("write your pallas kernel here")


# speedup vs baseline: 1.2894x; 1.2894x over previous
import jax
import jax.numpy as jnp
from jax.experimental import pallas as pl
from jax.experimental.pallas import tpu as pltpu

_T = 12
_B = 8
_N = 512
_D = 64
_P = 3
_O = 64


def _gg_kernel(xtp_ref, s_ref, w1t_ref, b1_ref, w2t_ref, b2_ref, out_ref):
    xb = xtp_ref[0]
    s = s_ref[...]
    y = jnp.dot(s, xb, preferred_element_type=jnp.float32)
    w1t = w1t_ref[...]
    w2t = w2t_ref[...]
    b1 = b1_ref[...]
    b2 = b2_ref[...]
    for i in range(_T):
        lo = i * _D
        hi = lo + _P * _D
        wy = y[:, lo:hi].astype(jnp.bfloat16)
        wx = xb[:, lo:hi]
        lin = jnp.dot(wy, w1t, preferred_element_type=jnp.float32) + b1
        inp2 = jnp.dot(wx, w2t, preferred_element_type=jnp.float32) + b2
        xh = lin[:, :_O]
        gate = lin[:, _O:]
        out_ref[i, 0] = (xh + inp2) * jax.nn.sigmoid(gate)


def kernel(x, supports, W1, b1, W2, b2):
    t, b, n, d = x.shape
    xt = jnp.transpose(x, (1, 2, 0, 3)).reshape(b, n, t * d)
    xtp = jnp.pad(xt, ((0, 0), (0, 0), (0, (_P - 1) * d))).astype(jnp.bfloat16)
    s16 = supports.astype(jnp.bfloat16)
    w1t = W1.T.astype(jnp.bfloat16)
    w2t = W2.T.astype(jnp.bfloat16)
    b1r = b1.reshape(1, -1)
    b2r = b2.reshape(1, -1)
    fd = (t + _P - 1) * d

    out = pl.pallas_call(
        _gg_kernel,
        grid=(b,),
        in_specs=[
            pl.BlockSpec((1, n, fd), lambda i: (i, 0, 0)),
            pl.BlockSpec((n, n), lambda i: (0, 0)),
            pl.BlockSpec((_P * d, 2 * _O), lambda i: (0, 0)),
            pl.BlockSpec((1, 2 * _O), lambda i: (0, 0)),
            pl.BlockSpec((_P * d, _O), lambda i: (0, 0)),
            pl.BlockSpec((1, _O), lambda i: (0, 0)),
        ],
        out_specs=pl.BlockSpec((t, 1, n, _O), lambda i: (0, i, 0, 0)),
        out_shape=jax.ShapeDtypeStruct((t, b, n, _O), jnp.float32),
    )(xtp, s16, w1t, b1r, w2t, b2r)
    return out


# speedup vs baseline: 1.4878x; 1.1539x over previous
import jax
import jax.numpy as jnp
from jax.experimental import pallas as pl
from jax.experimental.pallas import tpu as pltpu

_T = 12
_B = 8
_N = 512
_D = 64
_P = 3
_O = 64


def _gg_kernel(x_ref, s_ref, w1t_ref, b1_ref, w2t_ref, b2_ref, out_ref):
    pieces = [x_ref[t, 0].astype(jnp.bfloat16) for t in range(_T)]
    pieces.append(jnp.zeros((_N, (_P - 1) * _D), jnp.bfloat16))
    xb = jnp.concatenate(pieces, axis=-1)
    s = s_ref[...]
    y = jnp.dot(s, xb, preferred_element_type=jnp.float32)
    w1t = w1t_ref[...]
    w2t = w2t_ref[...]
    b1 = b1_ref[...]
    b2 = b2_ref[...]
    for i in range(_T):
        lo = i * _D
        hi = lo + _P * _D
        wy = y[:, lo:hi].astype(jnp.bfloat16)
        wx = xb[:, lo:hi]
        lin = jnp.dot(wy, w1t, preferred_element_type=jnp.float32) + b1
        inp2 = jnp.dot(wx, w2t, preferred_element_type=jnp.float32) + b2
        xh = lin[:, :_O]
        gate = lin[:, _O:]
        out_ref[i, 0] = (xh + inp2) * jax.nn.sigmoid(gate)


def kernel(x, supports, W1, b1, W2, b2):
    t, b, n, d = x.shape
    s16 = supports.astype(jnp.bfloat16)
    w1t = W1.T.astype(jnp.bfloat16)
    w2t = W2.T.astype(jnp.bfloat16)
    b1r = b1.reshape(1, -1)
    b2r = b2.reshape(1, -1)

    out = pl.pallas_call(
        _gg_kernel,
        grid=(b,),
        in_specs=[
            pl.BlockSpec((t, 1, n, d), lambda i: (0, i, 0, 0)),
            pl.BlockSpec((n, n), lambda i: (0, 0)),
            pl.BlockSpec((_P * d, 2 * _O), lambda i: (0, 0)),
            pl.BlockSpec((1, 2 * _O), lambda i: (0, 0)),
            pl.BlockSpec((_P * d, _O), lambda i: (0, 0)),
            pl.BlockSpec((1, _O), lambda i: (0, 0)),
        ],
        out_specs=pl.BlockSpec((t, 1, n, _O), lambda i: (0, i, 0, 0)),
        out_shape=jax.ShapeDtypeStruct((t, b, n, _O), jnp.float32),
    )(x, s16, w1t, b1r, w2t, b2r)
    return out


# speedup vs baseline: 4.0804x; 2.7425x over previous
import jax
import jax.numpy as jnp
from jax.experimental import pallas as pl
from jax.experimental.pallas import tpu as pltpu

_T = 12
_B = 8
_N = 512
_D = 64
_P = 3
_O = 64


def _gg_kernel(x_ref, st_ref, w1_ref, b1_ref, w2_ref, b2_ref, out_ref):
    xall = x_ref[:, 0].reshape(_T * _D, _N).astype(jnp.bfloat16)
    st = st_ref[...]
    yall = jnp.dot(xall, st, preferred_element_type=jnp.float32)
    yall = yall.astype(jnp.bfloat16)
    w1 = w1_ref[...]
    w2 = w2_ref[...]
    b1 = b1_ref[...]
    b2 = b2_ref[...]
    for i in range(_T):
        lo = i * _D
        hi = min(lo + _P * _D, _T * _D)
        k = hi - lo
        ywin = yall[lo:hi]
        xwin = xall[lo:hi]
        lin = jnp.dot(w1[:, :k], ywin, preferred_element_type=jnp.float32) + b1
        inp2 = jnp.dot(w2[:, :k], xwin, preferred_element_type=jnp.float32) + b2
        xh = lin[:_O]
        gate = lin[_O:]
        out_ref[i, 0] = (xh + inp2) * jax.nn.sigmoid(gate)


def kernel(x, supports, W1, b1, W2, b2):
    t, b, n, d = x.shape
    xp = jnp.transpose(x, (0, 1, 3, 2))
    st = supports.T.astype(jnp.bfloat16)
    w1 = W1.astype(jnp.bfloat16)
    w2 = W2.astype(jnp.bfloat16)
    b1c = b1.reshape(-1, 1)
    b2c = b2.reshape(-1, 1)

    out = pl.pallas_call(
        _gg_kernel,
        grid=(b,),
        in_specs=[
            pl.BlockSpec((t, 1, d, n), lambda i: (0, i, 0, 0)),
            pl.BlockSpec((n, n), lambda i: (0, 0)),
            pl.BlockSpec((2 * _O, _P * d), lambda i: (0, 0)),
            pl.BlockSpec((2 * _O, 1), lambda i: (0, 0)),
            pl.BlockSpec((_O, _P * d), lambda i: (0, 0)),
            pl.BlockSpec((_O, 1), lambda i: (0, 0)),
        ],
        out_specs=pl.BlockSpec((t, 1, _O, n), lambda i: (0, i, 0, 0)),
        out_shape=jax.ShapeDtypeStruct((t, b, _O, n), jnp.float32),
    )(xp, st, w1, b1c, w2, b2c)
    return jnp.transpose(out, (0, 1, 3, 2))


# speedup vs baseline: 5.3843x; 1.3196x over previous
import jax
import jax.numpy as jnp
from jax.experimental import pallas as pl
from jax.experimental.pallas import tpu as pltpu

_T = 12
_B = 8
_N = 512
_D = 64
_P = 3
_O = 64

_RHS_T = (((1,), (1,)), ((), ()))


def _gg_kernel(x_ref, s_ref, w1_ref, b1_ref, w2_ref, b2_ref, out_ref,
               s16_scr, w1_scr, w2_scr, b1_scr, b2_scr):
    @pl.when(pl.program_id(0) == 0)
    def _init():
        s16_scr[...] = s_ref[...].astype(jnp.bfloat16)
        w1_scr[...] = w1_ref[...].astype(jnp.bfloat16)
        w2_scr[...] = w2_ref[...].astype(jnp.bfloat16)
        b1_scr[...] = jnp.transpose(b1_ref[...], (1, 0))
        b2_scr[...] = jnp.transpose(b2_ref[...], (1, 0))

    xall = x_ref[:, 0].reshape(_T * _D, _N).astype(jnp.bfloat16)
    yall = jax.lax.dot_general(xall, s16_scr[...], _RHS_T,
                               preferred_element_type=jnp.float32)
    yall = yall.astype(jnp.bfloat16)
    w1 = w1_scr[...]
    w2 = w2_scr[...]
    b1 = b1_scr[...]
    b2 = b2_scr[...]
    for i in range(_T):
        lo = i * _D
        hi = min(lo + _P * _D, _T * _D)
        k = hi - lo
        ywin = yall[lo:hi]
        xwin = xall[lo:hi]
        lin = jnp.dot(w1[:, :k], ywin, preferred_element_type=jnp.float32) + b1
        inp2 = jnp.dot(w2[:, :k], xwin, preferred_element_type=jnp.float32) + b2
        xh = lin[:_O]
        gate = lin[_O:]
        out_ref[i, 0] = (xh + inp2) * jax.nn.sigmoid(gate)


def kernel(x, supports, W1, b1, W2, b2):
    t, b, n, d = x.shape
    xp = jnp.transpose(x, (0, 1, 3, 2))
    b1r = b1.reshape(1, -1)
    b2r = b2.reshape(1, -1)

    out = pl.pallas_call(
        _gg_kernel,
        grid=(b,),
        in_specs=[
            pl.BlockSpec((t, 1, d, n), lambda i: (0, i, 0, 0)),
            pl.BlockSpec((n, n), lambda i: (0, 0)),
            pl.BlockSpec((2 * _O, _P * d), lambda i: (0, 0)),
            pl.BlockSpec((1, 2 * _O), lambda i: (0, 0)),
            pl.BlockSpec((_O, _P * d), lambda i: (0, 0)),
            pl.BlockSpec((1, _O), lambda i: (0, 0)),
        ],
        out_specs=pl.BlockSpec((t, 1, _O, n), lambda i: (0, i, 0, 0)),
        out_shape=jax.ShapeDtypeStruct((t, b, _O, n), jnp.float32),
        scratch_shapes=[
            pltpu.VMEM((n, n), jnp.bfloat16),
            pltpu.VMEM((2 * _O, _P * d), jnp.bfloat16),
            pltpu.VMEM((_O, _P * d), jnp.bfloat16),
            pltpu.VMEM((2 * _O, 1), jnp.float32),
            pltpu.VMEM((_O, 1), jnp.float32),
        ],
    )(xp, supports, W1, b1r, W2, b2r)
    return jnp.transpose(out, (0, 1, 3, 2))
